# baseline (speedup 1.0000x reference)
import functools

import jax
import jax.numpy as jnp
from jax.experimental import pallas as pl
from jax.experimental.pallas import tpu as pltpu

N_TOKENS = 8192
D_MODEL = 768
N_EXPERTS = 8
BLOCK_T = 2048


def _moe_body(x_ref, gw_ref, ew_ref, eb_ref, o_ref, w_sm):
    e = pl.program_id(1)

    @pl.when(e == 0)
    def _gating():
        logits = jnp.dot(
            x_ref[...], gw_ref[...], preferred_element_type=jnp.float32
        )
        w = jax.nn.softmax(logits, axis=-1)
        w_sm[...] = w
        o_ref[...] = jnp.dot(
            w, eb_ref[...], preferred_element_type=jnp.float32
        ).astype(o_ref.dtype)

    y = jnp.dot(x_ref[...], ew_ref[e], preferred_element_type=jnp.float32)
    lane = jax.lax.broadcasted_iota(jnp.int32, (BLOCK_T, N_EXPERTS), 1)
    w_col = jnp.sum(jnp.where(lane == e, w_sm[...], 0.0), axis=1, keepdims=True)
    o_ref[...] += (w_col * y).astype(o_ref.dtype)


@functools.partial(jax.jit, static_argnames=("interpret",))
def kernel(inputs, gate_w, expert_w, expert_b, interpret=False):
    n_tokens, d_model = inputs.shape
    n_experts = expert_w.shape[0]
    grid = (n_tokens // BLOCK_T, n_experts)
    return pl.pallas_call(
        _moe_body,
        grid=grid,
        in_specs=[
            pl.BlockSpec((BLOCK_T, d_model), lambda i, e: (i, 0)),
            pl.BlockSpec((d_model, n_experts), lambda i, e: (0, 0)),
            pl.BlockSpec((n_experts, d_model, d_model), lambda i, e: (0, 0, 0)),
            pl.BlockSpec((n_experts, d_model), lambda i, e: (0, 0)),
        ],
        out_specs=pl.BlockSpec((BLOCK_T, d_model), lambda i, e: (i, 0)),
        out_shape=jax.ShapeDtypeStruct((n_tokens, d_model), inputs.dtype),
        scratch_shapes=[pltpu.VMEM((BLOCK_T, n_experts), jnp.float32)],
        interpret=interpret,
    )(inputs, gate_w, expert_w, expert_b)


# speedup vs baseline: 1.1445x; 1.1445x over previous
import functools

import jax
import jax.numpy as jnp
from jax.experimental import pallas as pl

N_TOKENS = 8192
D_MODEL = 768
N_EXPERTS = 8
BLOCK_T = 1024


def _moe_body(x_ref, gw_ref, ew_ref, eb_ref, o_ref):
    x = x_ref[...]
    logits = jnp.dot(x, gw_ref[...], preferred_element_type=jnp.float32)
    w = jax.nn.softmax(logits, axis=-1)
    acc = jnp.dot(w, eb_ref[...], preferred_element_type=jnp.float32)
    for e in range(N_EXPERTS):
        y = jnp.dot(x, ew_ref[e], preferred_element_type=jnp.float32)
        acc = acc + w[:, e : e + 1] * y
    o_ref[...] = acc.astype(o_ref.dtype)


@functools.partial(jax.jit, static_argnames=("interpret",))
def kernel(inputs, gate_w, expert_w, expert_b, interpret=False):
    n_tokens, d_model = inputs.shape
    n_experts = expert_w.shape[0]
    grid = (n_tokens // BLOCK_T,)
    return pl.pallas_call(
        _moe_body,
        grid=grid,
        in_specs=[
            pl.BlockSpec((BLOCK_T, d_model), lambda i: (i, 0)),
            pl.BlockSpec((d_model, n_experts), lambda i: (0, 0)),
            pl.BlockSpec((n_experts, d_model, d_model), lambda i: (0, 0, 0)),
            pl.BlockSpec((n_experts, d_model), lambda i: (0, 0)),
        ],
        out_specs=pl.BlockSpec((BLOCK_T, d_model), lambda i: (i, 0)),
        out_shape=jax.ShapeDtypeStruct((n_tokens, d_model), inputs.dtype),
        interpret=interpret,
    )(inputs, gate_w, expert_w, expert_b)
